# Initial kernel scaffold; baseline (speedup 1.0000x reference)
import jax
import jax.numpy as jnp
from jax.experimental import pallas as pl


def kernel(x, edge_index, batch, W1, b1, gamma1, beta1, W2, b2, gamma2, beta2, fc1_W, fc1_b, fc2_W, fc2_b):
    raise NotImplementedError("write your pallas kernel here")


# speedup vs baseline: 12.4394x; 12.4394x over previous
import functools

import jax
import jax.numpy as jnp
from jax import lax
from jax.experimental import pallas as pl
from jax.experimental.pallas import tpu as pltpu
from jax.experimental.pallas import tpu_sc as plsc

N = 10000
E = 320000
H = 128
G = 128

NC = 2
NS = 16
LANES = 16

EK = 80
TILE_E = E // (NC * NS)
NCHUNK = TILE_E // EK
NP = 10240
ROWS_T = NP // NS
ZR = 128

_mesh = plsc.VectorSubcoreMesh(
    core_axis_name="c", subcore_axis_name="s", num_cores=NC, num_subcores=NS
)


@functools.partial(
    pl.kernel,
    out_type=jax.ShapeDtypeStruct((NC, NP, H), jnp.float32),
    mesh=_mesh,
    scratch_types=[
        pltpu.VMEM((EK,), jnp.int32),
        pltpu.VMEM((EK, H), jnp.float32),
        pltpu.VMEM((ZR, H), jnp.float32),
        pltpu.VMEM_SHARED((NP, H), jnp.float32),
        pltpu.SemaphoreType.DMA,
    ],
)
def _sc_degree(dst_hbm, out_hbm, didx, ones_v, zbuf, acc, sem):
    c = lax.axis_index("c")
    s = lax.axis_index("s")

    def fill(k, _):
        ones_v[k // (H // LANES), pl.ds((k % (H // LANES)) * LANES, LANES)] = (
            jnp.full((LANES,), 1.0, jnp.float32)
        )
        return 0

    lax.fori_loop(0, EK * (H // LANES), fill, 0)

    def fillz(k, _):
        zbuf[k // (H // LANES), pl.ds((k % (H // LANES)) * LANES, LANES)] = (
            jnp.zeros((LANES,), jnp.float32)
        )
        return 0

    lax.fori_loop(0, ZR * (H // LANES), fillz, 0)

    def zero_acc(j, _):
        pltpu.sync_copy(zbuf, acc.at[pl.ds(s * ROWS_T + j * ZR, ZR)])
        return 0

    lax.fori_loop(0, ROWS_T // ZR, zero_acc, 0)
    plsc.subcore_barrier()

    ebase = (c * NS + s) * TILE_E

    def chunk(i, _):
        pltpu.sync_copy(dst_hbm.at[pl.ds(ebase + i * EK, EK)], didx)
        pltpu.sync_copy(ones_v, acc.at[didx], add=True)
        return 0

    lax.fori_loop(0, NCHUNK, chunk, 0)
    plsc.subcore_barrier()

    pltpu.sync_copy(
        acc.at[pl.ds(s * ROWS_T, ROWS_T)],
        out_hbm.at[c, pl.ds(s * ROWS_T, ROWS_T)],
    )


@functools.partial(
    pl.kernel,
    out_type=jax.ShapeDtypeStruct((NC, NP, H), jnp.float32),
    mesh=_mesh,
    scratch_types=[
        pltpu.VMEM((EK,), jnp.int32),
        pltpu.VMEM((EK,), jnp.int32),
        pltpu.VMEM((EK, H), jnp.float32),
        pltpu.VMEM((ZR, H), jnp.float32),
        pltpu.VMEM_SHARED((NP, H), jnp.float32),
        pltpu.SemaphoreType.DMA,
    ],
)
def _sc_agg(h_hbm, src_hbm, dst_hbm, out_hbm, sidx, didx, rows, zbuf, acc, sem):
    c = lax.axis_index("c")
    s = lax.axis_index("s")

    def fillz(k, _):
        zbuf[k // (H // LANES), pl.ds((k % (H // LANES)) * LANES, LANES)] = (
            jnp.zeros((LANES,), jnp.float32)
        )
        return 0

    lax.fori_loop(0, ZR * (H // LANES), fillz, 0)

    def zero_acc(j, _):
        pltpu.sync_copy(zbuf, acc.at[pl.ds(s * ROWS_T + j * ZR, ZR)])
        return 0

    lax.fori_loop(0, ROWS_T // ZR, zero_acc, 0)
    plsc.subcore_barrier()

    ebase = (c * NS + s) * TILE_E

    def chunk(i, _):
        pltpu.sync_copy(src_hbm.at[pl.ds(ebase + i * EK, EK)], sidx)
        pltpu.sync_copy(dst_hbm.at[pl.ds(ebase + i * EK, EK)], didx)
        pltpu.async_copy(h_hbm.at[sidx], rows, sem).wait()
        pltpu.sync_copy(rows, acc.at[didx], add=True)
        return 0

    lax.fori_loop(0, NCHUNK, chunk, 0)
    plsc.subcore_barrier()

    pltpu.sync_copy(
        acc.at[pl.ds(s * ROWS_T, ROWS_T)],
        out_hbm.at[c, pl.ds(s * ROWS_T, ROWS_T)],
    )


def _dinv_from(degp):
    deg = degp[0, :N] + degp[1, :N] + 1.0
    return lax.rsqrt(deg)


def _tc_mm1_body(x_ref, w_ref, degp_ref, o_ref):
    dinv = _dinv_from(degp_ref[...])
    h = jnp.dot(x_ref[...], w_ref[...], preferred_element_type=jnp.float32)
    o_ref[...] = h * dinv


def _bn_relu(aggp, hp, dinv, b, g, be):
    z = (aggp[0, :N] + aggp[1, :N] + hp) * dinv + b
    mu = jnp.mean(z, axis=0, keepdims=True)
    var = jnp.mean((z - mu) ** 2, axis=0, keepdims=True)
    return jnp.maximum((z - mu) * lax.rsqrt(var + 1e-5) * g + be, 0.0)


def _tc_bn_mm_body(aggp_ref, hp_ref, degp_ref, b_ref, g_ref, be_ref, w_ref, o_ref):
    dinv = _dinv_from(degp_ref[...])
    y = _bn_relu(aggp_ref[...], hp_ref[...], dinv, b_ref[...], g_ref[...], be_ref[...])
    o_ref[...] = jnp.dot(y, w_ref[...], preferred_element_type=jnp.float32) * dinv


def _tc_bn_pool_body(aggp_ref, hp_ref, degp_ref, b_ref, g_ref, be_ref,
                     batch_ref, fc1w_ref, fc1b_ref, fc2w_ref, fc2b_ref, o_ref):
    dinv = _dinv_from(degp_ref[...])
    y = _bn_relu(aggp_ref[...], hp_ref[...], dinv, b_ref[...], g_ref[...], be_ref[...])
    gid = lax.broadcasted_iota(jnp.int32, (G, N), 0)
    onehot_t = (batch_ref[...] == gid).astype(jnp.float32)
    sums = jnp.dot(onehot_t, y, preferred_element_type=jnp.float32)
    counts = jnp.sum(onehot_t, axis=1, keepdims=True)
    pooled = sums / jnp.maximum(counts, 1.0)
    a = jnp.maximum(
        jnp.dot(pooled, fc1w_ref[...], preferred_element_type=jnp.float32)
        + fc1b_ref[...],
        0.0,
    )
    o_ref[...] = (
        jnp.dot(a, fc2w_ref[...], preferred_element_type=jnp.float32) + fc2b_ref[...]
    )


def kernel(x, edge_index, batch, W1, b1, gamma1, beta1, W2, b2, gamma2, beta2,
           fc1_W, fc1_b, fc2_W, fc2_b):
    src = edge_index[0]
    dst = edge_index[1]

    degp = _sc_degree(dst)

    h1p = pl.pallas_call(
        _tc_mm1_body,
        out_shape=jax.ShapeDtypeStruct((N, H), jnp.float32),
    )(x, W1, degp)

    agg1 = _sc_agg(h1p, src, dst)

    h2p = pl.pallas_call(
        _tc_bn_mm_body,
        out_shape=jax.ShapeDtypeStruct((N, H), jnp.float32),
    )(agg1, h1p, degp, b1.reshape(1, H), gamma1.reshape(1, H),
      beta1.reshape(1, H), W2)

    agg2 = _sc_agg(h2p, src, dst)

    fc1w_p = jnp.pad(fc1_W, ((0, 0), (0, 128 - fc1_W.shape[1])))
    fc1b_p = jnp.pad(fc1_b, (0, 128 - fc1_b.shape[0])).reshape(1, 128)
    fc2w_p = jnp.pad(fc2_W, ((0, 128 - fc2_W.shape[0]), (0, 128 - fc2_W.shape[1])))
    fc2b_p = jnp.pad(fc2_b, (0, 128 - fc2_b.shape[0])).reshape(1, 128)

    out_p = pl.pallas_call(
        _tc_bn_pool_body,
        out_shape=jax.ShapeDtypeStruct((G, 128), jnp.float32),
    )(agg2, h2p, degp, b2.reshape(1, H), gamma2.reshape(1, H),
      beta2.reshape(1, H), batch.reshape(1, N), fc1w_p, fc1b_p, fc2w_p, fc2b_p)

    return out_p[:, : fc2_W.shape[1]]
